# baseline (speedup 1.0000x reference)
import jax
import jax.numpy as jnp
from jax.experimental import pallas as pl
from jax.experimental.pallas import tpu as pltpu

_BB = 128


def _body(xc_ref, xn_ref, ef_ref, w2s_ref, w2e_ref, w2v_ref, ws_ref,
          bvs2_ref, bs_ref, s_ref, outc_ref, outn_ref):
    xc = xc_ref[...]
    vc2 = jnp.dot(xc, w2v_ref[...],
                  preferred_element_type=jnp.float32) + bvs2_ref[...]
    outc_ref[...] = jnp.dot(xc, ws_ref[...],
                            preferred_element_type=jnp.float32) + bs_ref[...]
    nbr = jnp.dot(xn_ref[...], w2s_ref[...],
                  preferred_element_type=jnp.float32)
    nbr = nbr + jnp.dot(ef_ref[...], w2e_ref[...],
                        preferred_element_type=jnp.float32)
    nbr = nbr + jnp.dot(s_ref[...], vc2,
                        preferred_element_type=jnp.float32)
    outn_ref[...] = nbr


def kernel(central_node_features, neighbor_node_features, edge_features,
           Wq, bq, Wk, bk, Wv, bv, We, Wskip, bskip):
    b, n, d = neighbor_node_features.shape
    c = Wskip.shape[0]
    p = (n * d) // 128
    r = _BB * p

    xc = central_node_features.reshape(b, d)
    xn128 = neighbor_node_features.reshape(b * p, 128)
    ef128 = edge_features.reshape(b * p, 128)

    z = jnp.zeros((d, c), jnp.float32)
    ws_t = Wskip.T
    w2s = jnp.block([[ws_t, z], [z, ws_t]])
    we_t = We.T
    w2e = jnp.block([[we_t, z], [z, we_t]])
    w2v = jnp.concatenate([Wv.T, Wv.T], axis=1)
    bvs2 = jnp.tile(bv + bskip, 2).reshape(1, 128)
    bs = bskip.reshape(1, c)
    sel = (jnp.arange(r)[:, None] // p
           == jnp.arange(_BB)[None, :]).astype(jnp.float32)

    out_c, out_n = pl.pallas_call(
        _body,
        grid=(b // _BB,),
        in_specs=[
            pl.BlockSpec((_BB, d), lambda i: (i, 0)),
            pl.BlockSpec((r, 128), lambda i: (i, 0)),
            pl.BlockSpec((r, 128), lambda i: (i, 0)),
            pl.BlockSpec((128, 128), lambda i: (0, 0)),
            pl.BlockSpec((128, 128), lambda i: (0, 0)),
            pl.BlockSpec((d, 128), lambda i: (0, 0)),
            pl.BlockSpec((d, c), lambda i: (0, 0)),
            pl.BlockSpec((1, 128), lambda i: (0, 0)),
            pl.BlockSpec((1, c), lambda i: (0, 0)),
            pl.BlockSpec((r, _BB), lambda i: (0, 0)),
        ],
        out_specs=[
            pl.BlockSpec((_BB, c), lambda i: (i, 0)),
            pl.BlockSpec((r, 128), lambda i: (i, 0)),
        ],
        out_shape=[
            jax.ShapeDtypeStruct((b, c), jnp.float32),
            jax.ShapeDtypeStruct((b * p, 128), jnp.float32),
        ],
        compiler_params=pltpu.CompilerParams(
            dimension_semantics=("arbitrary",)),
    )(xc, xn128, ef128, w2s, w2e, w2v, ws_t, bvs2, bs, sel)

    out = jnp.concatenate([out_c[:, None, :], out_n.reshape(b, n, c)], axis=1)
    return out.reshape(b * (n + 1), c)


# speedup vs baseline: 1.3424x; 1.3424x over previous
import jax
import jax.numpy as jnp
from jax.experimental import pallas as pl
from jax.experimental.pallas import tpu as pltpu

_BB = 128
_NP = 56


def _body(xc_ref, xn_ref, ef_ref, ws_ref, wv_ref, we_ref, bvr_ref, bsr_ref,
          s_ref, out_ref, xn_pad, ef_pad, out_scr, sem_in, sem_out):
    n = xn_ref.shape[1]
    d = xn_ref.shape[2]
    c = ws_ref.shape[1]
    rows = _BB * _NP

    cin = pltpu.make_async_copy(xc_ref, xn_pad.at[:, 0:1, :], sem_in)
    nin = pltpu.make_async_copy(xn_ref, xn_pad.at[:, 1:n + 1, :], sem_in)
    ein = pltpu.make_async_copy(ef_ref, ef_pad.at[:, 1:n + 1, :], sem_in)
    cin.start()
    nin.start()
    ein.start()
    ef_pad[:, 0:1, :] = jnp.zeros((_BB, 1, d), jnp.float32)
    cin.wait()
    nin.wait()
    ein.wait()

    xnp = xn_pad[...].reshape(rows, d)
    efp = ef_pad[...].reshape(rows, d)
    xcv = xc_ref[...].reshape(_BB, d)

    vcb = jnp.dot(xcv, wv_ref[...], preferred_element_type=jnp.float32)
    vcb = vcb + bvr_ref[...]

    out_val = jnp.dot(xnp, ws_ref[...], preferred_element_type=jnp.float32)
    out_val = out_val + jnp.dot(efp, we_ref[...],
                                preferred_element_type=jnp.float32)
    out_val = out_val + jnp.dot(s_ref[...], vcb,
                                preferred_element_type=jnp.float32)
    out_val = out_val + bsr_ref[...]
    out_scr[...] = out_val.reshape(_BB, _NP, c)

    copies = [
        pltpu.make_async_copy(out_scr.at[s, 0:n + 1, :],
                              out_ref.at[pl.ds((n + 1) * s, n + 1), :],
                              sem_out)
        for s in range(_BB)
    ]
    for cp in copies:
        cp.start()
    for cp in copies:
        cp.wait()


def kernel(central_node_features, neighbor_node_features, edge_features,
           Wq, bq, Wk, bk, Wv, bv, We, Wskip, bskip):
    b, n, d = neighbor_node_features.shape
    c = Wskip.shape[0]
    m = b * (n + 1)
    rows = _BB * _NP

    ws_t = Wskip.T
    wv_t = Wv.T
    we_t = We.T
    bvr = (bv).reshape(1, c)
    bsr = bskip.reshape(1, c)
    t = jnp.arange(rows) % _NP
    sel = (((jnp.arange(rows) // _NP) == jnp.arange(_BB)[:, None]).T
           & (t >= 1)[:, None] & (t <= n)[:, None]).astype(jnp.float32)

    out = pl.pallas_call(
        _body,
        grid=(b // _BB,),
        in_specs=[
            pl.BlockSpec((_BB, 1, d), lambda i: (i, 0, 0)),
            pl.BlockSpec((_BB, n, d), lambda i: (i, 0, 0)),
            pl.BlockSpec((_BB, n, d), lambda i: (i, 0, 0)),
            pl.BlockSpec((d, c), lambda i: (0, 0)),
            pl.BlockSpec((d, c), lambda i: (0, 0)),
            pl.BlockSpec((d, c), lambda i: (0, 0)),
            pl.BlockSpec((1, c), lambda i: (0, 0)),
            pl.BlockSpec((1, c), lambda i: (0, 0)),
            pl.BlockSpec((rows, _BB), lambda i: (0, 0)),
        ],
        out_specs=pl.BlockSpec((_BB * (n + 1), c), lambda i: (i, 0)),
        out_shape=jax.ShapeDtypeStruct((m, c), jnp.float32),
        scratch_shapes=[
            pltpu.VMEM((_BB, _NP, d), jnp.float32),
            pltpu.VMEM((_BB, _NP, d), jnp.float32),
            pltpu.VMEM((_BB, _NP, c), jnp.float32),
            pltpu.SemaphoreType.DMA,
            pltpu.SemaphoreType.DMA,
        ],
        compiler_params=pltpu.CompilerParams(
            dimension_semantics=("arbitrary",)),
    )(central_node_features, neighbor_node_features, edge_features,
      ws_t, wv_t, we_t, bvr, bsr, sel)
    return out
